# baseline (speedup 1.0000x reference)
import numpy as np
import jax
import jax.numpy as jnp
from jax.experimental import pallas as pl
from jax.experimental.pallas import tpu as pltpu

B = 16384
N_FEAT = 26
EMB = 64
QUANTILES = np.array([-1.2816, -0.8416, -0.5244, -0.2533, 0.0,
                      0.2533, 0.5244, 0.8416, 1.2816], dtype=np.float32)
NP0, NB0 = 16, 20
NP1, NB1 = 32, 50
C0 = NP0 * (NB0 + 1)
C1 = NP1 * (NB1 + 1)
C = C0 + C1

NEG = np.float32(-3.0e38)


def _grid_pts(nb):
    res = 2.0 / nb
    return (np.linspace(-1.0, 1.0, nb + 1)[:-1] + 0.5 * res).astype(np.float32)


_TH = np.concatenate([
    np.repeat(np.concatenate([[NEG], _grid_pts(NB0)]).astype(np.float32), NP0),
    np.repeat(np.concatenate([[NEG], _grid_pts(NB1)]).astype(np.float32), NP1),
])

BB = 4096


def _body(x_ref, p_ref, th_ref, w0_ref, w1_ref, o_ref, dw_ref):
    @pl.when(pl.program_id(0) == 0)
    def _prep():
        w0 = w0_ref[...]
        w1 = w1_ref[...]
        dw0 = w0 - jnp.concatenate(
            [jnp.zeros((NP0, EMB), jnp.float32), w0[:C0 - NP0]], axis=0)
        dw1 = w1 - jnp.concatenate(
            [jnp.zeros((NP1, EMB), jnp.float32), w1[:C1 - NP1]], axis=0)
        dw = jnp.concatenate([dw0, dw1], axis=0)
        dwh = dw.astype(jnp.bfloat16)
        dwl = (dw - dwh.astype(jnp.float32)).astype(jnp.bfloat16)
        dw_ref[...] = jnp.concatenate([dwh, dwl], axis=1)

    x = x_ref[...]
    b = jnp.zeros_like(x)
    for q in QUANTILES:
        b += (x > q).astype(jnp.float32)
    xq = b / np.float32(10.0) - np.float32(0.5)
    n = jnp.sqrt(jnp.sum(xq * xq, axis=1, keepdims=True))
    xn = xq / jnp.maximum(n, np.float32(1e-12))
    z = jnp.dot(xn, p_ref[...], preferred_element_type=jnp.float32)
    z0 = z[:, :NP0]
    z1 = z[:, NP0:]
    zt = jnp.concatenate([z0] * (NB0 + 1) + [z1] * (NB1 + 1), axis=1)
    s = (zt > th_ref[...]).astype(jnp.bfloat16)
    acc2 = jnp.dot(s, dw_ref[...], preferred_element_type=jnp.float32)
    o_ref[...] = acc2[:, :EMB] + acc2[:, EMB:]


def kernel(f00, f01, f02, f03, f04, f05, f06, f07, f08, f09, f10, f11,
           f12, f13, f14, f15, f16, f17, f18, f19, f20, f21, f22, f23,
           f24, f25, proj0, proj1, w0, w1):
    feats = [f00, f01, f02, f03, f04, f05, f06, f07, f08, f09, f10, f11,
             f12, f13, f14, f15, f16, f17, f18, f19, f20, f21, f22, f23,
             f24, f25]
    x = jnp.concatenate(feats, axis=1)
    p = jnp.concatenate([proj0, proj1], axis=1)
    w0r = w0.reshape(NP0, NB0 + 1, EMB).transpose(1, 0, 2).reshape(C0, EMB)
    w1r = w1.reshape(NP1, NB1 + 1, EMB).transpose(1, 0, 2).reshape(C1, EMB)
    th = jnp.asarray(_TH)[None, :]

    out = pl.pallas_call(
        _body,
        grid=(B // BB,),
        in_specs=[
            pl.BlockSpec((BB, N_FEAT), lambda i: (i, 0)),
            pl.BlockSpec((N_FEAT, NP0 + NP1), lambda i: (0, 0)),
            pl.BlockSpec((1, C), lambda i: (0, 0)),
            pl.BlockSpec((C0, EMB), lambda i: (0, 0)),
            pl.BlockSpec((C1, EMB), lambda i: (0, 0)),
        ],
        out_specs=pl.BlockSpec((BB, EMB), lambda i: (i, 0)),
        out_shape=jax.ShapeDtypeStruct((B, EMB), jnp.float32),
        scratch_shapes=[pltpu.VMEM((C, 2 * EMB), jnp.bfloat16)],
    )(x, p, th, w0r, w1r)
    return out


# speedup vs baseline: 1.0022x; 1.0022x over previous
import numpy as np
import jax
import jax.numpy as jnp
from jax.experimental import pallas as pl
from jax.experimental.pallas import tpu as pltpu

B = 16384
N_FEAT = 26
EMB = 64
QUANTILES = np.array([-1.2816, -0.8416, -0.5244, -0.2533, 0.0,
                      0.2533, 0.5244, 0.8416, 1.2816], dtype=np.float32)
NP0, NB0 = 16, 20
NP1, NB1 = 32, 50
C0 = NP0 * (NB0 + 1)
C1 = NP1 * (NB1 + 1)
C = C0 + C1

NEG = np.float32(-3.0e38)


def _grid_pts(nb):
    res = 2.0 / nb
    return (np.linspace(-1.0, 1.0, nb + 1)[:-1] + 0.5 * res).astype(np.float32)


_TH = np.concatenate([
    np.repeat(np.concatenate([[NEG], _grid_pts(NB0)]).astype(np.float32), NP0),
    np.repeat(np.concatenate([[NEG], _grid_pts(NB1)]).astype(np.float32), NP1),
])

BB = 2048


def _body(x_ref, p_ref, th_ref, w0_ref, w1_ref, o_ref, dw_ref):
    @pl.when(pl.program_id(0) == 0)
    def _prep():
        w0 = w0_ref[...]
        w1 = w1_ref[...]
        dw0 = w0 - jnp.concatenate(
            [jnp.zeros((NP0, EMB), jnp.float32), w0[:C0 - NP0]], axis=0)
        dw1 = w1 - jnp.concatenate(
            [jnp.zeros((NP1, EMB), jnp.float32), w1[:C1 - NP1]], axis=0)
        dw = jnp.concatenate([dw0, dw1], axis=0)
        dwh = dw.astype(jnp.bfloat16)
        dwl = (dw - dwh.astype(jnp.float32)).astype(jnp.bfloat16)
        dw_ref[...] = jnp.concatenate([dwh, dwl], axis=1)

    x = x_ref[...]
    b = jnp.zeros_like(x)
    for q in QUANTILES:
        b += (x > q).astype(jnp.float32)
    xq = b / np.float32(10.0) - np.float32(0.5)
    n = jnp.sqrt(jnp.sum(xq * xq, axis=1, keepdims=True))
    xn = xq / jnp.maximum(n, np.float32(1e-12))
    z = jnp.dot(xn, p_ref[...], preferred_element_type=jnp.float32)
    z0 = z[:, :NP0]
    z1 = z[:, NP0:]
    zt = jnp.concatenate([z0] * (NB0 + 1) + [z1] * (NB1 + 1), axis=1)
    s = (zt > th_ref[...]).astype(jnp.bfloat16)
    acc2 = jnp.dot(s, dw_ref[...], preferred_element_type=jnp.float32)
    o_ref[...] = acc2[:, :EMB] + acc2[:, EMB:]


def kernel(f00, f01, f02, f03, f04, f05, f06, f07, f08, f09, f10, f11,
           f12, f13, f14, f15, f16, f17, f18, f19, f20, f21, f22, f23,
           f24, f25, proj0, proj1, w0, w1):
    feats = [f00, f01, f02, f03, f04, f05, f06, f07, f08, f09, f10, f11,
             f12, f13, f14, f15, f16, f17, f18, f19, f20, f21, f22, f23,
             f24, f25]
    x = jnp.concatenate(feats, axis=1)
    p = jnp.concatenate([proj0, proj1], axis=1)
    w0r = w0.reshape(NP0, NB0 + 1, EMB).transpose(1, 0, 2).reshape(C0, EMB)
    w1r = w1.reshape(NP1, NB1 + 1, EMB).transpose(1, 0, 2).reshape(C1, EMB)
    th = jnp.asarray(_TH)[None, :]

    out = pl.pallas_call(
        _body,
        grid=(B // BB,),
        in_specs=[
            pl.BlockSpec((BB, N_FEAT), lambda i: (i, 0)),
            pl.BlockSpec((N_FEAT, NP0 + NP1), lambda i: (0, 0)),
            pl.BlockSpec((1, C), lambda i: (0, 0)),
            pl.BlockSpec((C0, EMB), lambda i: (0, 0)),
            pl.BlockSpec((C1, EMB), lambda i: (0, 0)),
        ],
        out_specs=pl.BlockSpec((BB, EMB), lambda i: (i, 0)),
        out_shape=jax.ShapeDtypeStruct((B, EMB), jnp.float32),
        scratch_shapes=[pltpu.VMEM((C, 2 * EMB), jnp.bfloat16)],
    )(x, p, th, w0r, w1r)
    return out


# speedup vs baseline: 1.0064x; 1.0042x over previous
import numpy as np
import jax
import jax.numpy as jnp
from jax.experimental import pallas as pl
from jax.experimental.pallas import tpu as pltpu

B = 16384
N_FEAT = 26
EMB = 64
QUANTILES = np.array([-1.2816, -0.8416, -0.5244, -0.2533, 0.0,
                      0.2533, 0.5244, 0.8416, 1.2816], dtype=np.float32)
NP0, NB0 = 16, 20
NP1, NB1 = 32, 50
C0 = NP0 * (NB0 + 1)
C1 = NP1 * (NB1 + 1)
C = C0 + C1

NEG = np.float32(-3.0e38)


def _grid_pts(nb):
    res = 2.0 / nb
    return (np.linspace(-1.0, 1.0, nb + 1)[:-1] + 0.5 * res).astype(np.float32)


POS = np.float32(3.0e38)
NS0, NS1 = 3, 13
C0P, C1P = NS0 * 128, NS1 * 128
CP = C0P + C1P

_G0E = np.concatenate([[NEG], _grid_pts(NB0),
                       np.full(NS0 * 8 - (NB0 + 1), POS)]).astype(np.float32)
_G1E = np.concatenate([[NEG], _grid_pts(NB1),
                       np.full(NS1 * 4 - (NB1 + 1), POS)]).astype(np.float32)
_THP = np.empty((NS0 + NS1, 128), dtype=np.float32)
for _t in range(NS0):
    _THP[_t] = np.repeat(_G0E[_t * 8:(_t + 1) * 8], NP0)
for _t in range(NS1):
    _THP[NS0 + _t] = np.repeat(_G1E[_t * 4:(_t + 1) * 4], NP1)

BB = 2048


def _body(x_ref, p_ref, th_ref, w0_ref, w1_ref, o_ref, dw_ref):
    @pl.when(pl.program_id(0) == 0)
    def _prep():
        w0 = w0_ref[...]
        w1 = w1_ref[...]
        dw0 = w0 - jnp.concatenate(
            [jnp.zeros((NP0, EMB), jnp.float32), w0[:C0 - NP0]], axis=0)
        dw1 = w1 - jnp.concatenate(
            [jnp.zeros((NP1, EMB), jnp.float32), w1[:C1 - NP1]], axis=0)
        dw = jnp.concatenate([
            dw0, jnp.zeros((C0P - C0, EMB), jnp.float32),
            dw1, jnp.zeros((C1P - C1, EMB), jnp.float32)], axis=0)
        dwh = dw.astype(jnp.bfloat16)
        dwl = (dw - dwh.astype(jnp.float32)).astype(jnp.bfloat16)
        dw_ref[...] = jnp.concatenate([dwh, dwl], axis=1)

    x = x_ref[...]
    b = jnp.zeros_like(x)
    for q in QUANTILES:
        b += (x > q).astype(jnp.float32)
    xq = b / np.float32(10.0) - np.float32(0.5)
    n = jnp.sqrt(jnp.sum(xq * xq, axis=1, keepdims=True))
    xn = xq / jnp.maximum(n, np.float32(1e-12))
    z = jnp.dot(xn, p_ref[...], preferred_element_type=jnp.float32)
    z0 = z[:, :NP0]
    z1 = z[:, NP0:]
    zt0 = jnp.concatenate([z0] * 8, axis=1)
    zt1 = jnp.concatenate([z1] * 4, axis=1)
    th = th_ref[...]
    s = jnp.concatenate(
        [(zt0 > th[t:t + 1, :]).astype(jnp.bfloat16) for t in range(NS0)] +
        [(zt1 > th[NS0 + t:NS0 + t + 1, :]).astype(jnp.bfloat16)
         for t in range(NS1)], axis=1)
    acc2 = jnp.dot(s, dw_ref[...], preferred_element_type=jnp.float32)
    o_ref[...] = acc2[:, :EMB] + acc2[:, EMB:]


def kernel(f00, f01, f02, f03, f04, f05, f06, f07, f08, f09, f10, f11,
           f12, f13, f14, f15, f16, f17, f18, f19, f20, f21, f22, f23,
           f24, f25, proj0, proj1, w0, w1):
    feats = [f00, f01, f02, f03, f04, f05, f06, f07, f08, f09, f10, f11,
             f12, f13, f14, f15, f16, f17, f18, f19, f20, f21, f22, f23,
             f24, f25]
    x = jnp.concatenate(feats, axis=1)
    p = jnp.concatenate([proj0, proj1], axis=1)
    w0r = w0.reshape(NP0, NB0 + 1, EMB).transpose(1, 0, 2).reshape(C0, EMB)
    w1r = w1.reshape(NP1, NB1 + 1, EMB).transpose(1, 0, 2).reshape(C1, EMB)
    th = jnp.asarray(_THP)

    out = pl.pallas_call(
        _body,
        grid=(B // BB,),
        in_specs=[
            pl.BlockSpec((BB, N_FEAT), lambda i: (i, 0)),
            pl.BlockSpec((N_FEAT, NP0 + NP1), lambda i: (0, 0)),
            pl.BlockSpec((NS0 + NS1, 128), lambda i: (0, 0)),
            pl.BlockSpec((C0, EMB), lambda i: (0, 0)),
            pl.BlockSpec((C1, EMB), lambda i: (0, 0)),
        ],
        out_specs=pl.BlockSpec((BB, EMB), lambda i: (i, 0)),
        out_shape=jax.ShapeDtypeStruct((B, EMB), jnp.float32),
        scratch_shapes=[pltpu.VMEM((CP, 2 * EMB), jnp.bfloat16)],
    )(x, p, th, w0r, w1r)
    return out
